# Initial kernel scaffold; baseline (speedup 1.0000x reference)
import jax
import jax.numpy as jnp
from jax.experimental import pallas as pl


def kernel(x, edge_index, adj_values, W1, b1, W2, b2, gate):
    raise NotImplementedError("write your pallas kernel here")


# speedup vs baseline: 1.0646x; 1.0646x over previous
import functools

import jax
import jax.numpy as jnp
from jax.experimental import pallas as pl
from jax.experimental.pallas import tpu as pltpu

N = 10000
E = 160000
NFEAT = 256
NHID = 32
NCLASS = 16

_ROWS_BLK = 1000


def _node_prep_body(x_ref, w1_ref, fn_ref, xw_ref):
    x = x_ref[...]
    ss = jnp.sum(x * x, axis=1, keepdims=True)
    inv = 1.0 / (jnp.sqrt(ss) + 1e-12)
    fn_ref[...] = x * inv
    xw_ref[...] = jnp.dot(x, w1_ref[...], preferred_element_type=jnp.float32)


def _node_prep(x, W1):
    grid = (N // _ROWS_BLK,)
    return pl.pallas_call(
        _node_prep_body,
        grid=grid,
        in_specs=[
            pl.BlockSpec((_ROWS_BLK, NFEAT), lambda i: (i, 0)),
            pl.BlockSpec((NFEAT, NHID), lambda i: (0, 0)),
        ],
        out_specs=[
            pl.BlockSpec((_ROWS_BLK, NFEAT), lambda i: (i, 0)),
            pl.BlockSpec((_ROWS_BLK, NHID), lambda i: (i, 0)),
        ],
        out_shape=[
            jax.ShapeDtypeStruct((N, NFEAT), jnp.float32),
            jax.ShapeDtypeStruct((N, NHID), jnp.float32),
        ],
    )(x, W1)


def _att_pass(fea_n, row, col):
    sim = jnp.sum(fea_n[row] * fea_n[col], axis=1)
    sim = jnp.where(sim < 0.1, 0.0, sim)
    rs = jax.ops.segment_sum(sim, row, num_segments=N)
    inv_rs = jnp.where(rs > 0, 1.0 / jnp.where(rs > 0, rs, 1.0), 0.0)
    att = sim * inv_rs[row]
    degcnt = jax.ops.segment_sum((sim != 0.0).astype(jnp.float32), row,
                                 num_segments=N)
    lam = 1.0 / (degcnt + 1.0)
    ind = (rs > 0).astype(jnp.float32)
    return att, lam, ind


def kernel(x, edge_index, adj_values, W1, b1, W2, b2, gate):
    row, col = edge_index[0], edge_index[1]
    fn, xw = _node_prep(x, W1)

    att0, lam0, ind0 = _att_pass(fn, row, col)
    deg1 = ind0 + lam0
    dis1 = jax.lax.rsqrt(deg1)
    norm1 = dis1[row] * att0 * dis1[col]
    agg1 = jax.ops.segment_sum(norm1[:, None] * xw[row], col, num_segments=N)
    h = jax.nn.relu(agg1 + (dis1 * dis1 * lam0)[:, None] * xw + b1)

    hss = jnp.sum(h * h, axis=1, keepdims=True)
    hn = h * (1.0 / (jnp.sqrt(hss) + 1e-12))
    att1, lam1, ind1 = _att_pass(hn, row, col)

    g = gate[0]
    ve = g * att0 + (1.0 - g) * att1
    lamc = g * lam0 + (1.0 - g) * lam1
    deg2 = g * ind0 + (1.0 - g) * ind1 + lamc
    dis2 = jax.lax.rsqrt(deg2)
    norm2 = dis2[row] * ve * dis2[col]
    hw = jnp.matmul(h, W2)
    agg2 = jax.ops.segment_sum(norm2[:, None] * hw[row], col, num_segments=N)
    out = agg2 + (dis2 * dis2 * lamc)[:, None] * hw + b2
    return jax.nn.log_softmax(out, axis=1)


# speedup vs baseline: 9.8955x; 9.2953x over previous
import functools

import jax
import jax.numpy as jnp
from jax import lax
from jax.experimental import pallas as pl
from jax.experimental.pallas import tpu as pltpu
from jax.experimental.pallas import tpu_sc as plsc

N = 10000
E = 160000
NFEAT = 256
NHID = 32
NCLASS = 16

NC = 2
NS = 16
L = 16
NW = NC * NS
NP = 10240
NODES_PER_TILE = NP // NS
CHUNK = 64
NCHUNKS_TOTAL = E // CHUNK

_ROWS_BLK = 1000


def _mesh():
    return plsc.VectorSubcoreMesh(core_axis_name="c", subcore_axis_name="s",
                                  num_cores=NC, num_subcores=NS)


def _node_prep_body(x_ref, w1_ref, fn_ref, xw_ref):
    x = x_ref[...]
    ss = jnp.sum(x * x, axis=1, keepdims=True)
    inv = 1.0 / (jnp.sqrt(ss) + 1e-12)
    fn_ref[...] = x * inv
    xw_ref[...] = jnp.dot(x, w1_ref[...], preferred_element_type=jnp.float32)


def _node_prep(x, W1):
    return pl.pallas_call(
        _node_prep_body,
        grid=(N // _ROWS_BLK,),
        in_specs=[
            pl.BlockSpec((_ROWS_BLK, NFEAT), lambda i: (i, 0)),
            pl.BlockSpec((NFEAT, NHID), lambda i: (0, 0)),
        ],
        out_specs=[
            pl.BlockSpec((_ROWS_BLK, NFEAT), lambda i: (i, 0)),
            pl.BlockSpec((_ROWS_BLK, NHID), lambda i: (i, 0)),
        ],
        out_shape=[
            jax.ShapeDtypeStruct((N, NFEAT), jnp.float32),
            jax.ShapeDtypeStruct((N, NHID), jnp.float32),
        ],
    )(x, W1)


def _iota16():
    return lax.iota(jnp.int32, L)


def _worker_id():
    return lax.axis_index("s") * NC + lax.axis_index("c")


def _num_chunks(wid):
    per = NCHUNKS_TOTAL // NW
    return per + jnp.where(wid < NCHUNKS_TOTAL - per * NW, 1, 0)


def _sddmm_body(D, fn_hbm, row_hbm, col_hbm, sim_hbm, rsp_hbm, cntp_hbm,
                rowi_v, coli_v, rows_a, rows_b, sim_v, ind_v, accbuf,
                rs_sh, cnt_sh, sem):
    cid = lax.axis_index("c")
    sid = lax.axis_index("s")
    wid = _worker_id()
    iota = _iota16()
    zvec = jnp.zeros((L,), jnp.float32)

    for g in range(CHUNK // L):
        sim_v[pl.ds(g * L, L)] = zvec

    def zs(i, carry):
        base_n = sid * NODES_PER_TILE + i * CHUNK
        pltpu.sync_copy(sim_v, rs_sh.at[pl.ds(base_n, CHUNK)])
        pltpu.sync_copy(sim_v, cnt_sh.at[pl.ds(base_n, CHUNK)])
        return carry

    lax.fori_loop(0, NODES_PER_TILE // CHUNK, zs, 0)
    plsc.subcore_barrier()

    def chunk(j, carry):
        base = (wid + j * NW) * CHUNK
        pltpu.sync_copy(row_hbm.at[pl.ds(base, CHUNK)], rowi_v)
        pltpu.sync_copy(col_hbm.at[pl.ds(base, CHUNK)], coli_v)
        cp_a = pltpu.async_copy(fn_hbm.at[rowi_v], rows_a, sem)
        cp_b = pltpu.async_copy(fn_hbm.at[coli_v], rows_b, sem)
        cp_a.wait()
        cp_b.wait()

        def group(g, carry2):
            for j16 in range(L):
                k = g * L + j16
                acc = rows_a[k, pl.ds(0, L)] * rows_b[k, pl.ds(0, L)]
                for t in range(1, D // L):
                    acc = acc + (rows_a[k, pl.ds(t * L, L)]
                                 * rows_b[k, pl.ds(t * L, L)])
                accbuf[pl.ds(j16 * L, L)] = acc
            flat = iota * L
            sims = plsc.load_gather(accbuf, [flat])
            for i in range(1, L):
                sims = sims + plsc.load_gather(accbuf, [flat + i])
            sims = jnp.where(sims < 0.1, 0.0, sims)
            ind = jnp.where(sims != 0.0, 1.0, 0.0)
            sim_v[pl.ds(g * L, L)] = sims
            ind_v[pl.ds(g * L, L)] = ind
            return carry2

        lax.fori_loop(0, CHUNK // L, group, 0)
        pltpu.sync_copy(sim_v, sim_hbm.at[pl.ds(base, CHUNK)])
        pltpu.sync_copy(sim_v, rs_sh.at[rowi_v], add=True)
        pltpu.sync_copy(ind_v, cnt_sh.at[rowi_v], add=True)
        return carry

    lax.fori_loop(0, _num_chunks(wid), chunk, 0)
    plsc.subcore_barrier()
    base_n = sid * NODES_PER_TILE
    pltpu.sync_copy(rs_sh.at[pl.ds(base_n, NODES_PER_TILE)],
                    rsp_hbm.at[cid, pl.ds(base_n, NODES_PER_TILE)])
    pltpu.sync_copy(cnt_sh.at[pl.ds(base_n, NODES_PER_TILE)],
                    cntp_hbm.at[cid, pl.ds(base_n, NODES_PER_TILE)])


def _sddmm(fea_n, row, col, D):
    k = pl.kernel(
        functools.partial(_sddmm_body, D),
        out_type=[
            jax.ShapeDtypeStruct((E,), jnp.float32),
            jax.ShapeDtypeStruct((NC, NP), jnp.float32),
            jax.ShapeDtypeStruct((NC, NP), jnp.float32),
        ],
        mesh=_mesh(),
        compiler_params=pltpu.CompilerParams(needs_layout_passes=False, use_tc_tiling_on_sc=False),
        scratch_types=[
            pltpu.VMEM((CHUNK,), jnp.int32),
            pltpu.VMEM((CHUNK,), jnp.int32),
            pltpu.VMEM((CHUNK, D), jnp.float32),
            pltpu.VMEM((CHUNK, D), jnp.float32),
            pltpu.VMEM((CHUNK,), jnp.float32),
            pltpu.VMEM((CHUNK,), jnp.float32),
            pltpu.VMEM((L * L,), jnp.float32),
            pltpu.VMEM_SHARED((NP,), jnp.float32),
            pltpu.VMEM_SHARED((NP,), jnp.float32),
            pltpu.SemaphoreType.DMA,
        ],
    )
    return k(fea_n, row, col)


def _spmm_body(D, tab_hbm, row_hbm, col_hbm, sa_hbm, sb_hbm,
               wra_hbm, wrb_hbm, wc_hbm, accp_hbm,
               rowi_v, coli_v, sa_v, sb_v, msgs, wra_l, wrb_l, wc_l,
               zbuf, acc_sh, sem):
    cid = lax.axis_index("c")
    sid = lax.axis_index("s")
    wid = _worker_id()
    iota = _iota16()

    pltpu.sync_copy(wra_hbm, wra_l)
    pltpu.sync_copy(wrb_hbm, wrb_l)
    pltpu.sync_copy(wc_hbm, wc_l)

    zvec = jnp.zeros((L,), jnp.float32)
    for j in range(CHUNK):
        for t in range(D // L):
            zbuf[j, pl.ds(t * L, L)] = zvec

    def zs(i, carry):
        pltpu.sync_copy(zbuf, acc_sh.at[pl.ds(sid * NODES_PER_TILE + i * CHUNK, CHUNK)])
        return carry

    lax.fori_loop(0, NODES_PER_TILE // CHUNK, zs, 0)
    plsc.subcore_barrier()

    def chunk(j, carry):
        base = (wid + j * NW) * CHUNK
        pltpu.sync_copy(row_hbm.at[pl.ds(base, CHUNK)], rowi_v)
        pltpu.sync_copy(col_hbm.at[pl.ds(base, CHUNK)], coli_v)
        pltpu.sync_copy(sa_hbm.at[pl.ds(base, CHUNK)], sa_v)
        pltpu.sync_copy(sb_hbm.at[pl.ds(base, CHUNK)], sb_v)
        pltpu.async_copy(tab_hbm.at[rowi_v], msgs, sem).wait()

        def group(g, carry2):
            rowv = rowi_v[pl.ds(g * L, L)]
            colv = coli_v[pl.ds(g * L, L)]
            sa = sa_v[pl.ds(g * L, L)]
            sb = sb_v[pl.ds(g * L, L)]
            wra = plsc.load_gather(wra_l, [rowv])
            wrb = plsc.load_gather(wrb_l, [rowv])
            wc = plsc.load_gather(wc_l, [colv])
            wv = (wra * sa + wrb * sb) * wc
            for j16 in range(L):
                k = g * L + j16
                w = jnp.full((L,), wv[j16], jnp.float32)
                for t in range(D // L):
                    msgs[k, pl.ds(t * L, L)] = msgs[k, pl.ds(t * L, L)] * w
            return carry2

        lax.fori_loop(0, CHUNK // L, group, 0)
        pltpu.sync_copy(msgs, acc_sh.at[coli_v], add=True)
        return carry

    lax.fori_loop(0, _num_chunks(wid), chunk, 0)
    plsc.subcore_barrier()
    pltpu.sync_copy(acc_sh.at[pl.ds(sid * NODES_PER_TILE, NODES_PER_TILE)],
                    accp_hbm.at[cid, pl.ds(sid * NODES_PER_TILE, NODES_PER_TILE), :])


def _spmm(tab, row, col, sim_a, sim_b, wrow_a, wrow_b, wcol, D):
    k = pl.kernel(
        functools.partial(_spmm_body, D),
        out_type=[
            jax.ShapeDtypeStruct((NC, NP, D), jnp.float32),
        ],
        mesh=_mesh(),
        compiler_params=pltpu.CompilerParams(needs_layout_passes=False, use_tc_tiling_on_sc=False),
        scratch_types=[
            pltpu.VMEM((CHUNK,), jnp.int32),
            pltpu.VMEM((CHUNK,), jnp.int32),
            pltpu.VMEM((CHUNK,), jnp.float32),
            pltpu.VMEM((CHUNK,), jnp.float32),
            pltpu.VMEM((CHUNK, D), jnp.float32),
            pltpu.VMEM((NP,), jnp.float32),
            pltpu.VMEM((NP,), jnp.float32),
            pltpu.VMEM((NP,), jnp.float32),
            pltpu.VMEM((CHUNK, D), jnp.float32),
            pltpu.VMEM_SHARED((NP, D), jnp.float32),
            pltpu.SemaphoreType.DMA,
        ],
    )
    return k(tab, row, col, sim_a, sim_b, wrow_a, wrow_b, wcol)


def _pad_np(v):
    return jnp.pad(v, (0, NP - N))


def kernel(x, edge_index, adj_values, W1, b1, W2, b2, gate):
    row, col = edge_index[0], edge_index[1]
    fn, xw = _node_prep(x, W1)

    sim0, rsp0, cntp0 = _sddmm(fn, row, col, NFEAT)
    rs0 = rsp0[0, :N] + rsp0[1, :N]
    cnt0 = cntp0[0, :N] + cntp0[1, :N]
    inv_rs0 = jnp.where(rs0 > 0, 1.0 / jnp.where(rs0 > 0, rs0, 1.0), 0.0)
    lam0 = 1.0 / (cnt0 + 1.0)
    ind0 = (rs0 > 0).astype(jnp.float32)
    deg1 = ind0 + lam0
    dis1 = lax.rsqrt(deg1)

    zeros_e = jnp.zeros((E,), jnp.float32)
    zeros_n = jnp.zeros((NP,), jnp.float32)
    accp1 = _spmm(xw, row, col, sim0, zeros_e,
                  _pad_np(dis1 * inv_rs0), zeros_n, _pad_np(dis1), NHID)[0]
    agg1 = jnp.sum(accp1, axis=0)[:N]
    h = jax.nn.relu(agg1 + (dis1 * dis1 * lam0)[:, None] * xw + b1)

    hss = jnp.sum(h * h, axis=1, keepdims=True)
    hn = h * (1.0 / (jnp.sqrt(hss) + 1e-12))

    sim1, rsp1, cntp1 = _sddmm(hn, row, col, NHID)
    rs1 = rsp1[0, :N] + rsp1[1, :N]
    cnt1 = cntp1[0, :N] + cntp1[1, :N]
    inv_rs1 = jnp.where(rs1 > 0, 1.0 / jnp.where(rs1 > 0, rs1, 1.0), 0.0)
    lam1 = 1.0 / (cnt1 + 1.0)
    ind1 = (rs1 > 0).astype(jnp.float32)

    g = gate[0]
    lamc = g * lam0 + (1.0 - g) * lam1
    deg2 = g * ind0 + (1.0 - g) * ind1 + lamc
    dis2 = lax.rsqrt(deg2)

    hw = jnp.matmul(h, W2)
    accp2 = _spmm(hw, row, col, sim0, sim1,
                  _pad_np(g * dis2 * inv_rs0), _pad_np((1.0 - g) * dis2 * inv_rs1),
                  _pad_np(dis2), NCLASS)[0]
    agg2 = jnp.sum(accp2, axis=0)[:N]
    out = agg2 + (dis2 * dis2 * lamc)[:, None] * hw + b2
    return jax.nn.log_softmax(out, axis=1)


# speedup vs baseline: 17.6599x; 1.7846x over previous
import functools

import jax
import jax.numpy as jnp
from jax import lax
from jax.experimental import pallas as pl
from jax.experimental.pallas import tpu as pltpu
from jax.experimental.pallas import tpu_sc as plsc

N = 10000
E = 160000
NFEAT = 256
NHID = 32
NCLASS = 16

NC = 2
NS = 16
L = 16
NW = NC * NS
NP = 10240
NODES_PER_TILE = NP // NS
CHUNK = 64
NCHUNKS_TOTAL = E // CHUNK

_ROWS_BLK = 1000


def _mesh():
    return plsc.VectorSubcoreMesh(core_axis_name="c", subcore_axis_name="s",
                                  num_cores=NC, num_subcores=NS)


def _node_prep_body(x_ref, w1_ref, fn_ref, xw_ref):
    x = x_ref[...]
    ss = jnp.sum(x * x, axis=1, keepdims=True)
    inv = 1.0 / (jnp.sqrt(ss) + 1e-12)
    fn_ref[...] = x * inv
    xw_ref[...] = jnp.dot(x, w1_ref[...], preferred_element_type=jnp.float32)


def _node_prep(x, W1):
    return pl.pallas_call(
        _node_prep_body,
        grid=(N // _ROWS_BLK,),
        in_specs=[
            pl.BlockSpec((_ROWS_BLK, NFEAT), lambda i: (i, 0)),
            pl.BlockSpec((NFEAT, NHID), lambda i: (0, 0)),
        ],
        out_specs=[
            pl.BlockSpec((_ROWS_BLK, NFEAT), lambda i: (i, 0)),
            pl.BlockSpec((_ROWS_BLK, NHID), lambda i: (i, 0)),
        ],
        out_shape=[
            jax.ShapeDtypeStruct((N, NFEAT), jnp.float32),
            jax.ShapeDtypeStruct((N, NHID), jnp.float32),
        ],
    )(x, W1)


def _iota16():
    return lax.iota(jnp.int32, L)


def _worker_id():
    return lax.axis_index("s") * NC + lax.axis_index("c")


def _num_chunks(wid, nch_total):
    per = nch_total // NW
    return per + jnp.where(wid < nch_total - per * NW, 1, 0)


def _sddmm_body(D, CH, fn_hbm, row_hbm, col_hbm, sim_hbm, rsp_hbm, cntp_hbm,
                rowi0, coli0, rowi1, coli1, ra0, rb0, ra1, rb1,
                simv0, indv0, simv1, indv1, accbuf,
                rs_sh, cnt_sh, semg0, semg1):
    cid = lax.axis_index("c")
    sid = lax.axis_index("s")
    wid = _worker_id()
    iota = _iota16()
    zvec = jnp.zeros((L,), jnp.float32)
    nj = _num_chunks(wid, E // CH)
    bufs = ((rowi0, coli0, ra0, rb0, simv0, indv0, semg0),
            (rowi1, coli1, ra1, rb1, simv1, indv1, semg1))

    def stage_and_fire(c, b):
        rowi, coli, ra, rb, simv, indv, semg = bufs[b]
        base = (wid + c * NW) * CH
        pltpu.sync_copy(row_hbm.at[pl.ds(base, CH)], rowi)
        pltpu.sync_copy(col_hbm.at[pl.ds(base, CH)], coli)
        pltpu.async_copy(fn_hbm.at[rowi], ra, semg)
        pltpu.async_copy(fn_hbm.at[coli], rb, semg)

    def compute(c, b):
        rowi, coli, ra, rb, simv, indv, semg = bufs[b]
        pltpu.make_async_copy(fn_hbm.at[rowi], ra, semg).wait()
        pltpu.make_async_copy(fn_hbm.at[coli], rb, semg).wait()

        def group(g, carry2):
            for j16 in range(L):
                k = g * L + j16
                acc = ra[k, pl.ds(0, L)] * rb[k, pl.ds(0, L)]
                for t in range(1, D // L):
                    acc = acc + ra[k, pl.ds(t * L, L)] * rb[k, pl.ds(t * L, L)]
                accbuf[pl.ds(j16 * L, L)] = acc
            flat = iota * L
            sims = plsc.load_gather(accbuf, [flat])
            for i in range(1, L):
                sims = sims + plsc.load_gather(accbuf, [flat + i])
            sims = jnp.where(sims < 0.1, 0.0, sims)
            ind = jnp.where(sims != 0.0, 1.0, 0.0)
            simv[pl.ds(g * L, L)] = sims
            indv[pl.ds(g * L, L)] = ind
            return carry2

        lax.fori_loop(0, CH // L, group, 0)
        base = (wid + c * NW) * CH
        pltpu.sync_copy(simv, sim_hbm.at[pl.ds(base, CH)])
        pltpu.sync_copy(simv, rs_sh.at[rowi], add=True)
        pltpu.sync_copy(indv, cnt_sh.at[rowi], add=True)

    for g in range(CH // L):
        simv0[pl.ds(g * L, L)] = zvec

    def zs(i, carry):
        base_n = sid * NODES_PER_TILE + i * CH
        pltpu.sync_copy(simv0, rs_sh.at[pl.ds(base_n, CH)])
        pltpu.sync_copy(simv0, cnt_sh.at[pl.ds(base_n, CH)])
        return carry

    lax.fori_loop(0, NODES_PER_TILE // CH, zs, 0)
    stage_and_fire(0, 0)
    plsc.subcore_barrier()

    def iter2(i, carry):
        c0 = 2 * i

        @pl.when(c0 + 1 < nj)
        def _():
            stage_and_fire(c0 + 1, 1)

        compute(c0, 0)

        @pl.when(c0 + 1 < nj)
        def _():
            @pl.when(c0 + 2 < nj)
            def _():
                stage_and_fire(c0 + 2, 0)

            compute(c0 + 1, 1)

        return carry

    lax.fori_loop(0, (nj + 1) // 2, iter2, 0)
    plsc.subcore_barrier()
    base_n = sid * NODES_PER_TILE
    pltpu.sync_copy(rs_sh.at[pl.ds(base_n, NODES_PER_TILE)],
                    rsp_hbm.at[cid, pl.ds(base_n, NODES_PER_TILE)])
    pltpu.sync_copy(cnt_sh.at[pl.ds(base_n, NODES_PER_TILE)],
                    cntp_hbm.at[cid, pl.ds(base_n, NODES_PER_TILE)])


def _sddmm(fea_n, row, col, D, CH):
    k = pl.kernel(
        functools.partial(_sddmm_body, D, CH),
        out_type=[
            jax.ShapeDtypeStruct((E,), jnp.float32),
            jax.ShapeDtypeStruct((NC, NP), jnp.float32),
            jax.ShapeDtypeStruct((NC, NP), jnp.float32),
        ],
        mesh=_mesh(),
        compiler_params=pltpu.CompilerParams(needs_layout_passes=False, use_tc_tiling_on_sc=False),
        scratch_types=[
            pltpu.VMEM((CH,), jnp.int32),
            pltpu.VMEM((CH,), jnp.int32),
            pltpu.VMEM((CH,), jnp.int32),
            pltpu.VMEM((CH,), jnp.int32),
            pltpu.VMEM((CH, D), jnp.float32),
            pltpu.VMEM((CH, D), jnp.float32),
            pltpu.VMEM((CH, D), jnp.float32),
            pltpu.VMEM((CH, D), jnp.float32),
            pltpu.VMEM((CH,), jnp.float32),
            pltpu.VMEM((CH,), jnp.float32),
            pltpu.VMEM((CH,), jnp.float32),
            pltpu.VMEM((CH,), jnp.float32),
            pltpu.VMEM((L * L,), jnp.float32),
            pltpu.VMEM_SHARED((NP,), jnp.float32),
            pltpu.VMEM_SHARED((NP,), jnp.float32),
            pltpu.SemaphoreType.DMA,
            pltpu.SemaphoreType.DMA,
        ],
    )
    return k(fea_n, row, col)


def _spmm_body(D, CH, has_b, *refs):
    if has_b:
        (tab_hbm, row_hbm, col_hbm, sa_hbm, sb_hbm, wra_hbm, wrb_hbm, wc_hbm,
         accp_hbm,
         rowi0, coli0, rowi1, coli1, sav0, sav1, sbv0, sbv1, msgs0, msgs1,
         wra_l, wrb_l, wc_l, zbuf, acc_sh,
         semg0, semg1) = refs
        bufs = ((rowi0, coli0, sav0, sbv0, msgs0, semg0),
                (rowi1, coli1, sav1, sbv1, msgs1, semg1))
    else:
        (tab_hbm, row_hbm, col_hbm, sa_hbm, wra_hbm, wc_hbm,
         accp_hbm,
         rowi0, coli0, rowi1, coli1, sav0, sav1, msgs0, msgs1,
         wra_l, wc_l, zbuf, acc_sh,
         semg0, semg1) = refs
        bufs = ((rowi0, coli0, sav0, None, msgs0, semg0),
                (rowi1, coli1, sav1, None, msgs1, semg1))
    cid = lax.axis_index("c")
    sid = lax.axis_index("s")
    wid = _worker_id()
    nj = _num_chunks(wid, E // CH)

    pltpu.sync_copy(wra_hbm, wra_l)
    if has_b:
        pltpu.sync_copy(wrb_hbm, wrb_l)
    pltpu.sync_copy(wc_hbm, wc_l)

    def stage_and_fire(c, b):
        rowi, coli, sav, sbv, msgs, semg = bufs[b]
        base = (wid + c * NW) * CH
        pltpu.sync_copy(row_hbm.at[pl.ds(base, CH)], rowi)
        pltpu.sync_copy(col_hbm.at[pl.ds(base, CH)], coli)
        pltpu.sync_copy(sa_hbm.at[pl.ds(base, CH)], sav)
        if has_b:
            pltpu.sync_copy(sb_hbm.at[pl.ds(base, CH)], sbv)
        pltpu.async_copy(tab_hbm.at[rowi], msgs, semg)

    def compute(c, b):
        rowi, coli, sav, sbv, msgs, semg = bufs[b]
        pltpu.make_async_copy(tab_hbm.at[rowi], msgs, semg).wait()

        def group(g, carry2):
            rowv = rowi[pl.ds(g * L, L)]
            colv = coli[pl.ds(g * L, L)]
            wra = plsc.load_gather(wra_l, [rowv])
            wc = plsc.load_gather(wc_l, [colv])
            wv = wra * sav[pl.ds(g * L, L)]
            if has_b:
                wrb = plsc.load_gather(wrb_l, [rowv])
                wv = wv + wrb * sbv[pl.ds(g * L, L)]
            wv = wv * wc
            for j16 in range(L):
                k = g * L + j16
                w = jnp.full((L,), wv[j16], jnp.float32)
                for t in range(D // L):
                    msgs[k, pl.ds(t * L, L)] = msgs[k, pl.ds(t * L, L)] * w
            return carry2

        lax.fori_loop(0, CH // L, group, 0)
        pltpu.sync_copy(msgs, acc_sh.at[coli], add=True)

    zvec = jnp.zeros((L,), jnp.float32)
    for j in range(64):
        for t in range(D // L):
            zbuf[j, pl.ds(t * L, L)] = zvec

    def zs(i, carry):
        pltpu.sync_copy(zbuf, acc_sh.at[pl.ds(sid * NODES_PER_TILE + i * 64, 64)])
        return carry

    lax.fori_loop(0, NODES_PER_TILE // 64, zs, 0)
    stage_and_fire(0, 0)
    plsc.subcore_barrier()

    def iter2(i, carry):
        c0 = 2 * i

        @pl.when(c0 + 1 < nj)
        def _():
            stage_and_fire(c0 + 1, 1)

        compute(c0, 0)

        @pl.when(c0 + 1 < nj)
        def _():
            @pl.when(c0 + 2 < nj)
            def _():
                stage_and_fire(c0 + 2, 0)

            compute(c0 + 1, 1)

        return carry

    lax.fori_loop(0, (nj + 1) // 2, iter2, 0)
    plsc.subcore_barrier()
    pltpu.sync_copy(acc_sh.at[pl.ds(sid * NODES_PER_TILE, NODES_PER_TILE)],
                    accp_hbm.at[cid, pl.ds(sid * NODES_PER_TILE, NODES_PER_TILE), :])


def _spmm(tab, row, col, sims, wrows, wcol, D, CH):
    has_b = len(sims) == 2
    st = [
        pltpu.VMEM((CH,), jnp.int32),
        pltpu.VMEM((CH,), jnp.int32),
        pltpu.VMEM((CH,), jnp.int32),
        pltpu.VMEM((CH,), jnp.int32),
        pltpu.VMEM((CH,), jnp.float32),
        pltpu.VMEM((CH,), jnp.float32),
    ]
    if has_b:
        st += [pltpu.VMEM((CH,), jnp.float32), pltpu.VMEM((CH,), jnp.float32)]
    st += [
        pltpu.VMEM((CH, D), jnp.float32),
        pltpu.VMEM((CH, D), jnp.float32),
        pltpu.VMEM((NP,), jnp.float32),
    ]
    if has_b:
        st += [pltpu.VMEM((NP,), jnp.float32)]
    st += [
        pltpu.VMEM((NP,), jnp.float32),
        pltpu.VMEM((64, D), jnp.float32),
        pltpu.VMEM_SHARED((NP, D), jnp.float32),
        pltpu.SemaphoreType.DMA,
        pltpu.SemaphoreType.DMA,
    ]
    k = pl.kernel(
        functools.partial(_spmm_body, D, CH, has_b),
        out_type=[
            jax.ShapeDtypeStruct((NC, NP, D), jnp.float32),
        ],
        mesh=_mesh(),
        compiler_params=pltpu.CompilerParams(needs_layout_passes=False, use_tc_tiling_on_sc=False),
        scratch_types=st,
    )
    return k(tab, row, col, *sims, *wrows, wcol)


def _pad_np(v):
    return jnp.pad(v, (0, NP - N))


def kernel(x, edge_index, adj_values, W1, b1, W2, b2, gate):
    row, col = edge_index[0], edge_index[1]
    fn, xw = _node_prep(x, W1)

    sim0, rsp0, cntp0 = _sddmm(fn, row, col, NFEAT, 64)
    rs0 = rsp0[0, :N] + rsp0[1, :N]
    cnt0 = cntp0[0, :N] + cntp0[1, :N]
    inv_rs0 = jnp.where(rs0 > 0, 1.0 / jnp.where(rs0 > 0, rs0, 1.0), 0.0)
    lam0 = 1.0 / (cnt0 + 1.0)
    ind0 = (rs0 > 0).astype(jnp.float32)
    deg1 = ind0 + lam0
    dis1 = lax.rsqrt(deg1)

    accp1 = _spmm(xw, row, col, [sim0], [_pad_np(dis1 * inv_rs0)],
                  _pad_np(dis1), NHID, 128)[0]
    agg1 = jnp.sum(accp1, axis=0)[:N]
    h = jax.nn.relu(agg1 + (dis1 * dis1 * lam0)[:, None] * xw + b1)

    hss = jnp.sum(h * h, axis=1, keepdims=True)
    hn = h * (1.0 / (jnp.sqrt(hss) + 1e-12))

    sim1, rsp1, cntp1 = _sddmm(hn, row, col, NHID, 128)
    rs1 = rsp1[0, :N] + rsp1[1, :N]
    cnt1 = cntp1[0, :N] + cntp1[1, :N]
    inv_rs1 = jnp.where(rs1 > 0, 1.0 / jnp.where(rs1 > 0, rs1, 1.0), 0.0)
    lam1 = 1.0 / (cnt1 + 1.0)
    ind1 = (rs1 > 0).astype(jnp.float32)

    g = gate[0]
    lamc = g * lam0 + (1.0 - g) * lam1
    deg2 = g * ind0 + (1.0 - g) * ind1 + lamc
    dis2 = lax.rsqrt(deg2)

    hw = jnp.matmul(h, W2)
    accp2 = _spmm(hw, row, col, [sim0, sim1],
                  [_pad_np(g * dis2 * inv_rs0),
                   _pad_np((1.0 - g) * dis2 * inv_rs1)],
                  _pad_np(dis2), NCLASS, 128)[0]
    agg2 = jnp.sum(accp2, axis=0)[:N]
    out = agg2 + (dis2 * dis2 * lamc)[:, None] * hw + b2
    return jax.nn.log_softmax(out, axis=1)


# speedup vs baseline: 18.0866x; 1.0242x over previous
import functools

import jax
import jax.numpy as jnp
from jax import lax
from jax.experimental import pallas as pl
from jax.experimental.pallas import tpu as pltpu
from jax.experimental.pallas import tpu_sc as plsc

N = 10000
E = 160000
NFEAT = 256
NHID = 32
NCLASS = 16

NC = 2
NS = 16
L = 16
NW = NC * NS
NP = 10240
NODES_PER_TILE = NP // NS
CHUNK = 64
NCHUNKS_TOTAL = E // CHUNK

_ROWS_BLK = 1000


def _mesh():
    return plsc.VectorSubcoreMesh(core_axis_name="c", subcore_axis_name="s",
                                  num_cores=NC, num_subcores=NS)


def _node_prep_body(x_ref, w1_ref, fn_ref, xw_ref):
    x = x_ref[...]
    ss = jnp.sum(x * x, axis=1, keepdims=True)
    inv = 1.0 / (jnp.sqrt(ss) + 1e-12)
    fn_ref[...] = x * inv
    xw_ref[...] = jnp.dot(x, w1_ref[...], preferred_element_type=jnp.float32)


def _node_prep(x, W1):
    return pl.pallas_call(
        _node_prep_body,
        grid=(N // _ROWS_BLK,),
        in_specs=[
            pl.BlockSpec((_ROWS_BLK, NFEAT), lambda i: (i, 0)),
            pl.BlockSpec((NFEAT, NHID), lambda i: (0, 0)),
        ],
        out_specs=[
            pl.BlockSpec((_ROWS_BLK, NFEAT), lambda i: (i, 0)),
            pl.BlockSpec((_ROWS_BLK, NHID), lambda i: (i, 0)),
        ],
        out_shape=[
            jax.ShapeDtypeStruct((N, NFEAT), jnp.float32),
            jax.ShapeDtypeStruct((N, NHID), jnp.float32),
        ],
    )(x, W1)


def _iota16():
    return lax.iota(jnp.int32, L)


def _worker_id():
    return lax.axis_index("s") * NC + lax.axis_index("c")


def _num_chunks(wid, nch_total):
    per = nch_total // NW
    return per + jnp.where(wid < nch_total - per * NW, 1, 0)


def _sddmm_body(D, CH, fn_hbm, row_hbm, col_hbm, sim_hbm, rsp_hbm, cntp_hbm,
                rowi0, coli0, rowi1, coli1, ra0, rb0, ra1, rb1,
                simv0, indv0, simv1, indv1, accbuf,
                rs_sh, cnt_sh, semg0, semg1):
    cid = lax.axis_index("c")
    sid = lax.axis_index("s")
    wid = _worker_id()
    iota = _iota16()
    zvec = jnp.zeros((L,), jnp.float32)
    nj = _num_chunks(wid, E // CH)
    bufs = ((rowi0, coli0, ra0, rb0, simv0, indv0, semg0),
            (rowi1, coli1, ra1, rb1, simv1, indv1, semg1))

    def stage_and_fire(c, b):
        rowi, coli, ra, rb, simv, indv, semg = bufs[b]
        base = (wid + c * NW) * CH
        pltpu.sync_copy(row_hbm.at[pl.ds(base, CH)], rowi)
        pltpu.sync_copy(col_hbm.at[pl.ds(base, CH)], coli)
        pltpu.async_copy(fn_hbm.at[rowi], ra, semg)
        pltpu.async_copy(fn_hbm.at[coli], rb, semg)

    def compute(c, b):
        rowi, coli, ra, rb, simv, indv, semg = bufs[b]
        pltpu.make_async_copy(fn_hbm.at[rowi], ra, semg).wait()
        pltpu.make_async_copy(fn_hbm.at[coli], rb, semg).wait()

        def group(g, carry2):
            for j16 in range(L):
                k = g * L + j16
                acc = ra[k, pl.ds(0, L)] * rb[k, pl.ds(0, L)]
                for t in range(1, D // L):
                    acc = acc + ra[k, pl.ds(t * L, L)] * rb[k, pl.ds(t * L, L)]
                accbuf[pl.ds(j16 * L, L)] = acc
            flat = iota * L
            sims = plsc.load_gather(accbuf, [flat])
            for i in range(1, L):
                sims = sims + plsc.load_gather(accbuf, [flat + i])
            sims = jnp.where(sims < 0.1, 0.0, sims)
            ind = jnp.where(sims != 0.0, 1.0, 0.0)
            simv[pl.ds(g * L, L)] = sims
            indv[pl.ds(g * L, L)] = ind
            return carry2

        lax.fori_loop(0, CH // L, group, 0)
        base = (wid + c * NW) * CH
        pltpu.sync_copy(simv, sim_hbm.at[pl.ds(base, CH)])
        pltpu.sync_copy(simv, rs_sh.at[rowi], add=True)
        pltpu.sync_copy(indv, cnt_sh.at[rowi], add=True)

    for g in range(CH // L):
        simv0[pl.ds(g * L, L)] = zvec

    def zs(i, carry):
        base_n = sid * NODES_PER_TILE + i * CH
        pltpu.sync_copy(simv0, rs_sh.at[pl.ds(base_n, CH)])
        pltpu.sync_copy(simv0, cnt_sh.at[pl.ds(base_n, CH)])
        return carry

    lax.fori_loop(0, NODES_PER_TILE // CH, zs, 0)
    stage_and_fire(0, 0)
    plsc.subcore_barrier()

    def iter2(i, carry):
        c0 = 2 * i

        @pl.when(c0 + 1 < nj)
        def _():
            stage_and_fire(c0 + 1, 1)

        compute(c0, 0)

        @pl.when(c0 + 1 < nj)
        def _():
            @pl.when(c0 + 2 < nj)
            def _():
                stage_and_fire(c0 + 2, 0)

            compute(c0 + 1, 1)

        return carry

    lax.fori_loop(0, (nj + 1) // 2, iter2, 0)
    plsc.subcore_barrier()
    base_n = sid * NODES_PER_TILE
    pltpu.sync_copy(rs_sh.at[pl.ds(base_n, NODES_PER_TILE)],
                    rsp_hbm.at[cid, pl.ds(base_n, NODES_PER_TILE)])
    pltpu.sync_copy(cnt_sh.at[pl.ds(base_n, NODES_PER_TILE)],
                    cntp_hbm.at[cid, pl.ds(base_n, NODES_PER_TILE)])


def _sddmm(fea_n, row, col, D, CH):
    k = pl.kernel(
        functools.partial(_sddmm_body, D, CH),
        out_type=[
            jax.ShapeDtypeStruct((E,), jnp.float32),
            jax.ShapeDtypeStruct((NC, NP), jnp.float32),
            jax.ShapeDtypeStruct((NC, NP), jnp.float32),
        ],
        mesh=_mesh(),
        compiler_params=pltpu.CompilerParams(needs_layout_passes=False, use_tc_tiling_on_sc=False),
        scratch_types=[
            pltpu.VMEM((CH,), jnp.int32),
            pltpu.VMEM((CH,), jnp.int32),
            pltpu.VMEM((CH,), jnp.int32),
            pltpu.VMEM((CH,), jnp.int32),
            pltpu.VMEM((CH, D), jnp.float32),
            pltpu.VMEM((CH, D), jnp.float32),
            pltpu.VMEM((CH, D), jnp.float32),
            pltpu.VMEM((CH, D), jnp.float32),
            pltpu.VMEM((CH,), jnp.float32),
            pltpu.VMEM((CH,), jnp.float32),
            pltpu.VMEM((CH,), jnp.float32),
            pltpu.VMEM((CH,), jnp.float32),
            pltpu.VMEM((L * L,), jnp.float32),
            pltpu.VMEM_SHARED((NP,), jnp.float32),
            pltpu.VMEM_SHARED((NP,), jnp.float32),
            pltpu.SemaphoreType.DMA,
            pltpu.SemaphoreType.DMA,
        ],
    )
    return k(fea_n, row, col)


def _spmm_body(D, CH, has_b, *refs):
    if has_b:
        (tab_hbm, row_hbm, col_hbm, sa_hbm, sb_hbm, wra_hbm, wrb_hbm, wc_hbm,
         accp_hbm,
         rowi0, coli0, rowi1, coli1, sav0, sav1, sbv0, sbv1, msgs0, msgs1,
         wra_l, wrb_l, wc_l, zbuf, acc_sh,
         semg0, semg1) = refs
        bufs = ((rowi0, coli0, sav0, sbv0, msgs0, semg0),
                (rowi1, coli1, sav1, sbv1, msgs1, semg1))
    else:
        (tab_hbm, row_hbm, col_hbm, sa_hbm, wra_hbm, wc_hbm,
         accp_hbm,
         rowi0, coli0, rowi1, coli1, sav0, sav1, msgs0, msgs1,
         wra_l, wc_l, zbuf, acc_sh,
         semg0, semg1) = refs
        bufs = ((rowi0, coli0, sav0, None, msgs0, semg0),
                (rowi1, coli1, sav1, None, msgs1, semg1))
    cid = lax.axis_index("c")
    sid = lax.axis_index("s")
    wid = _worker_id()
    nj = _num_chunks(wid, E // CH)

    pltpu.sync_copy(wra_hbm, wra_l)
    if has_b:
        pltpu.sync_copy(wrb_hbm, wrb_l)
    pltpu.sync_copy(wc_hbm, wc_l)

    def stage_and_fire(c, b):
        rowi, coli, sav, sbv, msgs, semg = bufs[b]
        base = (wid + c * NW) * CH
        pltpu.sync_copy(row_hbm.at[pl.ds(base, CH)], rowi)
        pltpu.sync_copy(col_hbm.at[pl.ds(base, CH)], coli)
        pltpu.sync_copy(sa_hbm.at[pl.ds(base, CH)], sav)
        if has_b:
            pltpu.sync_copy(sb_hbm.at[pl.ds(base, CH)], sbv)
        pltpu.async_copy(tab_hbm.at[rowi], msgs, semg)

    def compute(c, b):
        rowi, coli, sav, sbv, msgs, semg = bufs[b]
        pltpu.make_async_copy(tab_hbm.at[rowi], msgs, semg).wait()

        def group(g, carry2):
            rowv = rowi[pl.ds(g * L, L)]
            colv = coli[pl.ds(g * L, L)]
            wra = plsc.load_gather(wra_l, [rowv])
            wc = plsc.load_gather(wc_l, [colv])
            wv = wra * sav[pl.ds(g * L, L)]
            if has_b:
                wrb = plsc.load_gather(wrb_l, [rowv])
                wv = wv + wrb * sbv[pl.ds(g * L, L)]
            wv = wv * wc
            for j16 in range(L):
                k = g * L + j16
                w = jnp.full((L,), wv[j16], jnp.float32)
                for t in range(D // L):
                    msgs[k, pl.ds(t * L, L)] = msgs[k, pl.ds(t * L, L)] * w
            return carry2

        lax.fori_loop(0, CH // L, group, 0)
        pltpu.sync_copy(msgs, acc_sh.at[coli], add=True)

    zvec = jnp.zeros((L,), jnp.float32)
    for j in range(64):
        for t in range(D // L):
            zbuf[j, pl.ds(t * L, L)] = zvec

    def zs(i, carry):
        pltpu.sync_copy(zbuf, acc_sh.at[pl.ds(sid * NODES_PER_TILE + i * 64, 64)])
        return carry

    lax.fori_loop(0, NODES_PER_TILE // 64, zs, 0)
    stage_and_fire(0, 0)
    plsc.subcore_barrier()

    def iter2(i, carry):
        c0 = 2 * i

        @pl.when(c0 + 1 < nj)
        def _():
            stage_and_fire(c0 + 1, 1)

        compute(c0, 0)

        @pl.when(c0 + 1 < nj)
        def _():
            @pl.when(c0 + 2 < nj)
            def _():
                stage_and_fire(c0 + 2, 0)

            compute(c0 + 1, 1)

        return carry

    lax.fori_loop(0, (nj + 1) // 2, iter2, 0)
    plsc.subcore_barrier()
    pltpu.sync_copy(acc_sh.at[pl.ds(sid * NODES_PER_TILE, NODES_PER_TILE)],
                    accp_hbm.at[cid, pl.ds(sid * NODES_PER_TILE, NODES_PER_TILE), :])


def _spmm(tab, row, col, sims, wrows, wcol, D, CH):
    has_b = len(sims) == 2
    st = [
        pltpu.VMEM((CH,), jnp.int32),
        pltpu.VMEM((CH,), jnp.int32),
        pltpu.VMEM((CH,), jnp.int32),
        pltpu.VMEM((CH,), jnp.int32),
        pltpu.VMEM((CH,), jnp.float32),
        pltpu.VMEM((CH,), jnp.float32),
    ]
    if has_b:
        st += [pltpu.VMEM((CH,), jnp.float32), pltpu.VMEM((CH,), jnp.float32)]
    st += [
        pltpu.VMEM((CH, D), jnp.float32),
        pltpu.VMEM((CH, D), jnp.float32),
        pltpu.VMEM((NP,), jnp.float32),
    ]
    if has_b:
        st += [pltpu.VMEM((NP,), jnp.float32)]
    st += [
        pltpu.VMEM((NP,), jnp.float32),
        pltpu.VMEM((64, D), jnp.float32),
        pltpu.VMEM_SHARED((NP, D), jnp.float32),
        pltpu.SemaphoreType.DMA,
        pltpu.SemaphoreType.DMA,
    ]
    k = pl.kernel(
        functools.partial(_spmm_body, D, CH, has_b),
        out_type=[
            jax.ShapeDtypeStruct((NC, NP, D), jnp.float32),
        ],
        mesh=_mesh(),
        compiler_params=pltpu.CompilerParams(needs_layout_passes=False, use_tc_tiling_on_sc=False),
        scratch_types=st,
    )
    return k(tab, row, col, *sims, *wrows, wcol)


def _pad_np(v):
    return jnp.pad(v, (0, NP - N))


def kernel(x, edge_index, adj_values, W1, b1, W2, b2, gate):
    row, col = edge_index[0], edge_index[1]
    fn, xw = _node_prep(x, W1)

    sim0, rsp0, cntp0 = _sddmm(fn, row, col, NFEAT, 80)
    rs0 = rsp0[0, :N] + rsp0[1, :N]
    cnt0 = cntp0[0, :N] + cntp0[1, :N]
    inv_rs0 = jnp.where(rs0 > 0, 1.0 / jnp.where(rs0 > 0, rs0, 1.0), 0.0)
    lam0 = 1.0 / (cnt0 + 1.0)
    ind0 = (rs0 > 0).astype(jnp.float32)
    deg1 = ind0 + lam0
    dis1 = lax.rsqrt(deg1)

    accp1 = _spmm(xw, row, col, [sim0], [_pad_np(dis1 * inv_rs0)],
                  _pad_np(dis1), NHID, 128)[0]
    agg1 = jnp.sum(accp1, axis=0)[:N]
    h = jax.nn.relu(agg1 + (dis1 * dis1 * lam0)[:, None] * xw + b1)

    hss = jnp.sum(h * h, axis=1, keepdims=True)
    hn = h * (1.0 / (jnp.sqrt(hss) + 1e-12))

    sim1, rsp1, cntp1 = _sddmm(hn, row, col, NHID, 128)
    rs1 = rsp1[0, :N] + rsp1[1, :N]
    cnt1 = cntp1[0, :N] + cntp1[1, :N]
    inv_rs1 = jnp.where(rs1 > 0, 1.0 / jnp.where(rs1 > 0, rs1, 1.0), 0.0)
    lam1 = 1.0 / (cnt1 + 1.0)
    ind1 = (rs1 > 0).astype(jnp.float32)

    g = gate[0]
    lamc = g * lam0 + (1.0 - g) * lam1
    deg2 = g * ind0 + (1.0 - g) * ind1 + lamc
    dis2 = lax.rsqrt(deg2)

    hw = jnp.matmul(h, W2)
    accp2 = _spmm(hw, row, col, [sim0, sim1],
                  [_pad_np(g * dis2 * inv_rs0),
                   _pad_np((1.0 - g) * dis2 * inv_rs1)],
                  _pad_np(dis2), NCLASS, 128)[0]
    agg2 = jnp.sum(accp2, axis=0)[:N]
    out = agg2 + (dis2 * dis2 * lamc)[:, None] * hw + b2
    return jax.nn.log_softmax(out, axis=1)
